# Initial kernel scaffold; baseline (speedup 1.0000x reference)
import jax
import jax.numpy as jnp
from jax.experimental import pallas as pl


def kernel(tokens, W_pos):
    raise NotImplementedError("write your pallas kernel here")


# speedup vs baseline: 1.4507x; 1.4507x over previous
import jax
import jax.numpy as jnp
from jax.experimental import pallas as pl


def _body(w_ref, o_ref):
    o_ref[...] = jnp.broadcast_to(w_ref[...][None, :, :], o_ref.shape)


def kernel(tokens, W_pos):
    batch, seq = tokens.shape
    d_model = W_pos.shape[-1]
    blk = 512
    return pl.pallas_call(
        _body,
        grid=(seq // blk,),
        in_specs=[pl.BlockSpec((blk, d_model), lambda i: (i, 0))],
        out_specs=pl.BlockSpec((batch, blk, d_model), lambda i: (0, i, 0)),
        out_shape=jax.ShapeDtypeStruct((batch, seq, d_model), W_pos.dtype),
    )(W_pos)


# speedup vs baseline: 1.5062x; 1.0383x over previous
import jax
import jax.numpy as jnp
from jax.experimental import pallas as pl


def _body(w_ref, o_ref):
    o_ref[...] = jnp.broadcast_to(w_ref[...][None, :, :], o_ref.shape)


def kernel(tokens, W_pos):
    batch, seq = tokens.shape
    d_model = W_pos.shape[-1]
    blk = 1024
    return pl.pallas_call(
        _body,
        grid=(seq // blk,),
        in_specs=[pl.BlockSpec((blk, d_model), lambda i: (i, 0))],
        out_specs=pl.BlockSpec((batch, blk, d_model), lambda i: (0, i, 0)),
        out_shape=jax.ShapeDtypeStruct((batch, seq, d_model), W_pos.dtype),
    )(W_pos)
